# baseline (speedup 1.0000x reference)
import functools

import jax
import jax.numpy as jnp
from jax import lax
from jax.experimental import pallas as pl
from jax.experimental.pallas import tpu as pltpu
from jax.experimental.pallas import tpu_sc as plsc

N = 20000
MAX_OUT = 100
IOU_THRESH = 0.7

L = 16
NSUB = 16
NPAD = 20480
SHARD = NPAD // NSUB
NSL = SHARD // L
OUTPAD = 512


def _nms_sc_body(x1h, y1h, x2h, y2h, sch, outh,
                 x1v, y1v, x2v, y2v, scv, arv, wkv, ixv,
                 tbl_sh, tblv, stage, scr, outv):
    sid = lax.axis_index("s")
    cid = lax.axis_index("c")
    base = sid * SHARD

    pltpu.sync_copy(x1h.at[pl.ds(base, SHARD)], x1v)
    pltpu.sync_copy(y1h.at[pl.ds(base, SHARD)], y1v)
    pltpu.sync_copy(x2h.at[pl.ds(base, SHARD)], x2v)
    pltpu.sync_copy(y2h.at[pl.ds(base, SHARD)], y2v)
    pltpu.sync_copy(sch.at[pl.ds(base, SHARD)], scv)

    ninf = jnp.float32(-jnp.inf)
    ii = lax.iota(jnp.int32, L)
    iif = ii.astype(jnp.float32)
    zz = ii * 0
    base_f = jnp.broadcast_to(base, (L,)).astype(jnp.float32)

    def xlane_argmax(v, ix):
        ix_s, v_s = plsc.sort_key_val(ix, v, descending=False)
        v_s2, ix_s2 = plsc.sort_key_val(v_s, ix_s, descending=True)
        scr[pl.ds(0, L)] = v_s2
        scr[pl.ds(L, L)] = ix_s2
        return plsc.load_gather(scr, [zz]), plsc.load_gather(scr, [zz + L])

    def init_j(j, st):
        bv, bi = st
        s = pl.ds(j * L, L)
        a, b, c, d, sc = x1v[s], y1v[s], x2v[s], y2v[s], scv[s]
        arv[s] = jnp.maximum(c - a, 0.0) * jnp.maximum(d - b, 0.0)
        valid = (c > a + 1.0) & (d > b + 1.0)
        w = jnp.where(valid, sc, ninf)
        wkv[s] = w
        ix = base_f + jnp.broadcast_to(j * L, (L,)).astype(jnp.float32) + iif
        ixv[s] = ix
        m = w > bv
        return jnp.where(m, w, bv), jnp.where(m, ix, bi)

    bv0 = jnp.broadcast_to(ninf, (L,))
    bi0 = base_f + iif
    bv, bi = lax.fori_loop(0, NSL, init_j, (bv0, bi0))

    def round_body(i, st):
        bv, bi = st
        lmax, li = xlane_argmax(bv, bi)
        loc = (li - base_f).astype(jnp.int32)
        pub = jnp.where(ii == 0, lmax, jnp.float32(0.0))
        pub = jnp.where(ii == 1, li, pub)
        pub = jnp.where(ii == 2, plsc.load_gather(x1v, [loc]), pub)
        pub = jnp.where(ii == 3, plsc.load_gather(y1v, [loc]), pub)
        pub = jnp.where(ii == 4, plsc.load_gather(x2v, [loc]), pub)
        pub = jnp.where(ii == 5, plsc.load_gather(y2v, [loc]), pub)
        pub = jnp.where(ii == 6, plsc.load_gather(scv, [loc]), pub)
        stage[...] = pub
        pltpu.sync_copy(stage, tbl_sh.at[pl.ds(sid * L, L)])
        plsc.subcore_barrier()
        pltpu.sync_copy(tbl_sh, tblv)
        plsc.subcore_barrier()

        rows = ii * L
        vals = plsc.load_gather(tblv, [rows])
        idxs = plsc.load_gather(tblv, [rows + 1])
        _, gbi = xlane_argmax(vals, idxs)
        wrow = (gbi.astype(jnp.int32) // SHARD) * L
        bx1 = plsc.load_gather(tblv, [wrow + 2])
        by1 = plsc.load_gather(tblv, [wrow + 3])
        bx2 = plsc.load_gather(tblv, [wrow + 4])
        by2 = plsc.load_gather(tblv, [wrow + 5])
        bsc = plsc.load_gather(tblv, [wrow + 6])
        bar = jnp.maximum(bx2 - bx1, 0.0) * jnp.maximum(by2 - by1, 0.0)

        def sup_j(j, st2):
            nbv, nbi = st2
            s = pl.ds(j * L, L)
            ix1 = jnp.maximum(x1v[s], bx1)
            iy1 = jnp.maximum(y1v[s], by1)
            ix2 = jnp.minimum(x2v[s], bx2)
            iy2 = jnp.minimum(y2v[s], by2)
            inter = jnp.maximum(ix2 - ix1, 0.0) * jnp.maximum(iy2 - iy1, 0.0)
            iou = inter / (arv[s] + bar - inter + 1e-8)
            ixs = ixv[s]
            w = jnp.where((iou > IOU_THRESH) | (ixs == gbi), ninf, wkv[s])
            wkv[s] = w
            m = w > nbv
            return jnp.where(m, w, nbv), jnp.where(m, ixs, nbi)

        nbv, nbi = lax.fori_loop(0, NSL, sup_j, (bv0, bi0))

        row = jnp.where(ii == 0, bx1, jnp.float32(0.0))
        row = jnp.where(ii == 1, by1, row)
        row = jnp.where(ii == 2, bx2, row)
        row = jnp.where(ii == 3, by2, row)
        row = jnp.where(ii == 4, bsc, row)
        plsc.store_scatter(outv, [i * 5 + ii], row, mask=ii < 5)
        return nbv, nbi

    lax.fori_loop(0, MAX_OUT, round_body, (bv, bi))

    @pl.when((sid == 0) & (cid == 0))
    def _():
        pltpu.sync_copy(outv, outh)


def _make_nms_sc():
    mesh = plsc.VectorSubcoreMesh(
        core_axis_name="c", subcore_axis_name="s", num_cores=2, num_subcores=NSUB
    )
    return pl.kernel(
        _nms_sc_body,
        out_type=jax.ShapeDtypeStruct((OUTPAD,), jnp.float32),
        mesh=mesh,
        compiler_params=pltpu.CompilerParams(needs_layout_passes=False),
        scratch_types=[
            pltpu.VMEM((SHARD,), jnp.float32),
            pltpu.VMEM((SHARD,), jnp.float32),
            pltpu.VMEM((SHARD,), jnp.float32),
            pltpu.VMEM((SHARD,), jnp.float32),
            pltpu.VMEM((SHARD,), jnp.float32),
            pltpu.VMEM((SHARD,), jnp.float32),
            pltpu.VMEM((SHARD,), jnp.float32),
            pltpu.VMEM((SHARD,), jnp.float32),
            pltpu.VMEM_SHARED((NSUB * L,), jnp.float32),
            pltpu.VMEM((NSUB * L,), jnp.float32),
            pltpu.VMEM((L,), jnp.float32),
            pltpu.VMEM((2 * L,), jnp.float32),
            pltpu.VMEM((OUTPAD,), jnp.float32),
        ],
    )


def kernel(boxes, scores):
    pad = NPAD - N
    x1 = jnp.pad(boxes[:, 0], (0, pad))
    y1 = jnp.pad(boxes[:, 1], (0, pad))
    x2 = jnp.pad(boxes[:, 2], (0, pad))
    y2 = jnp.pad(boxes[:, 3], (0, pad))
    sc = jnp.pad(scores, (0, pad))
    out = _make_nms_sc()(x1, y1, x2, y2, sc)
    return out[: MAX_OUT * 5].reshape(MAX_OUT, 5)
